# baseline (speedup 1.0000x reference)
import functools

import jax
import jax.numpy as jnp
from jax import lax
from jax.experimental import pallas as pl
from jax.experimental.pallas import tpu as pltpu
from jax.experimental.pallas import tpu_sc as plsc

BATCH = 4096
SEQ = 200
D_MODEL = 64
SCALE = float(D_MODEL) ** 0.5
NC, NS = 2, 16
NW = NC * NS
BW = BATCH // NW
NPAIR = SEQ // 2
LANES = 128


def _sc_body(xt_hbm, table_hbm, mid_hbm,
             ibuf, rows0, rows1, gsem0, gsem1, ssem0, ssem1):
    wid = lax.axis_index("s") * NC + lax.axis_index("c")
    b0 = wid * BW

    pltpu.sync_copy(xt_hbm.at[:, pl.ds(b0, BW)], ibuf)

    def gather(t, rbuf, sem):
        return pltpu.async_copy(table_hbm.at[ibuf.at[t]], rbuf, sem)

    def store(t, rbuf, sem):
        pltpu.async_copy(
            rbuf, mid_hbm.at[t, pl.ds(b0, BW), pl.ds(0, D_MODEL)], sem)

    def store_wait(rbuf, sem):
        pltpu.make_async_copy(
            rbuf, mid_hbm.at[0, pl.ds(b0, BW), pl.ds(0, D_MODEL)], sem).wait()

    def pair(p, carry):
        t0 = 2 * p
        t1 = 2 * p + 1

        @pl.when(p > 0)
        def _():
            store_wait(rows0, ssem0)

        g0 = gather(t0, rows0, gsem0)

        @pl.when(p > 0)
        def _():
            store_wait(rows1, ssem1)

        g1 = gather(t1, rows1, gsem1)

        g0.wait()
        store(t0, rows0, ssem0)
        g1.wait()
        store(t1, rows1, ssem1)
        return carry

    lax.fori_loop(0, NPAIR, pair, 0)
    store_wait(rows0, ssem0)
    store_wait(rows1, ssem1)


def _tc_body(m_ref, o_ref):
    blk = m_ref[0]
    o_ref[0] = blk.T * SCALE


@jax.jit
def _run(x, table):
    mesh = plsc.VectorSubcoreMesh(core_axis_name="c", subcore_axis_name="s")
    sc = functools.partial(
        pl.kernel,
        out_type=jax.ShapeDtypeStruct((SEQ, BATCH, D_MODEL), jnp.float32),
        mesh=mesh,
        compiler_params=pltpu.CompilerParams(
            use_tc_tiling_on_sc=False, needs_layout_passes=False),
        scratch_types=[
            pltpu.VMEM((SEQ, BW), jnp.int32),
            pltpu.VMEM((BW, D_MODEL), jnp.float32),
            pltpu.VMEM((BW, D_MODEL), jnp.float32),
            pltpu.SemaphoreType.DMA,
            pltpu.SemaphoreType.DMA,
            pltpu.SemaphoreType.DMA,
            pltpu.SemaphoreType.DMA,
        ],
    )(_sc_body)
    mid = sc(x.T, table)

    out_phys = pl.pallas_call(
        _tc_body,
        grid=(SEQ,),
        in_specs=[
            pl.BlockSpec((1, BATCH, D_MODEL), lambda t: (t, 0, 0)),
        ],
        out_specs=pl.BlockSpec((1, D_MODEL, BATCH), lambda t: (t, 0, 0)),
        out_shape=jax.ShapeDtypeStruct((SEQ, D_MODEL, BATCH), jnp.float32),
        compiler_params=pltpu.CompilerParams(
            dimension_semantics=("arbitrary",)),
    )(mid)
    return out_phys.transpose(2, 0, 1)


def kernel(x, table):
    return _run(x, table)


# speedup vs baseline: 1.2773x; 1.2773x over previous
import functools

import jax
import jax.numpy as jnp
from jax import lax
from jax.experimental import pallas as pl
from jax.experimental.pallas import tpu as pltpu
from jax.experimental.pallas import tpu_sc as plsc

BATCH = 4096
SEQ = 200
D_MODEL = 64
SCALE = float(D_MODEL) ** 0.5
NC, NS = 2, 16
NW = NC * NS
BW = BATCH // NW
NPAIR = SEQ // 2
LANES = 128


def _sc_body(xt_hbm, table_hbm, mid_hbm,
             ibuf, rows0, rows1, gsem0, gsem1, ssem0, ssem1):
    wid = lax.axis_index("s") * NC + lax.axis_index("c")
    b0 = wid * BW

    pltpu.sync_copy(xt_hbm.at[:, pl.ds(b0, BW)], ibuf)

    def gather(t, rbuf, sem):
        return pltpu.async_copy(table_hbm.at[ibuf.at[t]], rbuf, sem)

    def store(t, rbuf, sem):
        pltpu.async_copy(
            rbuf, mid_hbm.at[t, pl.ds(b0, BW), pl.ds(0, D_MODEL)], sem)

    def store_wait(rbuf, sem):
        pltpu.make_async_copy(
            rbuf, mid_hbm.at[0, pl.ds(b0, BW), pl.ds(0, D_MODEL)], sem).wait()

    def pair(p, carry):
        t0 = 2 * p
        t1 = 2 * p + 1

        @pl.when(p > 0)
        def _():
            store_wait(rows0, ssem0)

        g0 = gather(t0, rows0, gsem0)

        @pl.when(p > 0)
        def _():
            store_wait(rows1, ssem1)

        g1 = gather(t1, rows1, gsem1)

        g0.wait()
        store(t0, rows0, ssem0)
        g1.wait()
        store(t1, rows1, ssem1)
        return carry

    lax.fori_loop(0, NPAIR, pair, 0)
    store_wait(rows0, ssem0)
    store_wait(rows1, ssem1)


def _tc_body(m_ref, o_ref):
    blk = m_ref[0]
    o_ref[0] = blk[:, :D_MODEL].T * SCALE


@jax.jit
def _run(x, table):
    mesh = plsc.VectorSubcoreMesh(core_axis_name="c", subcore_axis_name="s")
    sc = functools.partial(
        pl.kernel,
        out_type=jax.ShapeDtypeStruct((SEQ, BATCH, LANES), jnp.float32),
        mesh=mesh,
        compiler_params=pltpu.CompilerParams(
            use_tc_tiling_on_sc=False, needs_layout_passes=False),
        scratch_types=[
            pltpu.VMEM((SEQ, BW), jnp.int32),
            pltpu.VMEM((BW, D_MODEL), jnp.float32),
            pltpu.VMEM((BW, D_MODEL), jnp.float32),
            pltpu.SemaphoreType.DMA,
            pltpu.SemaphoreType.DMA,
            pltpu.SemaphoreType.DMA,
            pltpu.SemaphoreType.DMA,
        ],
    )(_sc_body)
    mid = sc(x.T, table)

    out_phys = pl.pallas_call(
        _tc_body,
        grid=(SEQ,),
        in_specs=[
            pl.BlockSpec((1, BATCH, LANES), lambda t: (t, 0, 0)),
        ],
        out_specs=pl.BlockSpec((1, D_MODEL, BATCH), lambda t: (t, 0, 0)),
        out_shape=jax.ShapeDtypeStruct((SEQ, D_MODEL, BATCH), jnp.float32),
        compiler_params=pltpu.CompilerParams(
            dimension_semantics=("arbitrary",)),
    )(mid)
    return out_phys.transpose(2, 0, 1)


def kernel(x, table):
    return _run(x, table)
